# baseline (speedup 1.0000x reference)
import jax
import jax.numpy as jnp
from jax import lax
from jax.experimental import pallas as pl
from jax.experimental.pallas import tpu as pltpu
from jax.experimental.pallas import tpu_sc as plsc

_N = 10000
_E = 320000
_D = 128
_DW = _D // 2
_R = 8
_NC = 2
_NS = 16
_NW = _NC * _NS
_EPW = _E // _NW
_C = 80
_NCHUNK = _EPW // _C
_NBUF = 4
_NLOOPS = (_NCHUNK - (_NBUF - 1)) // _NBUF

_GATHER_DNUMS = lax.GatherDimensionNumbers(
    offset_dims=(), collapsed_slice_dims=(0,), start_index_map=(0,))


def _lane_shuffle(v, idx):
    return lax.gather(v, idx[:, None], _GATHER_DNUMS, slice_sizes=(1,),
                      mode=lax.GatherScatterMode.PROMISE_IN_BOUNDS)


def _bf16_bits(x):
    u = lax.bitcast_convert_type(x, jnp.int32)
    rounded = u + 0x7FFF + lax.bitwise_and(lax.shift_right_logical(u, 16), 1)
    return lax.shift_right_logical(rounded, 16)


def _pack_words(x):
    lo = _bf16_bits(x[:, :_DW])
    hi = _bf16_bits(x[:, _DW:])
    return lax.bitwise_or(lo, lax.shift_left(hi, 16))


def _scale_body(h_ref, w_ref, g_ref, hb_ref):
    r = pl.program_id(0)
    zeros = jnp.zeros((_N, _DW), jnp.int32)
    g = h_ref[...] * w_ref[pl.ds(r, 1), :]
    g_ref[...] = jnp.concatenate([_pack_words(g), zeros], axis=1)[None]
    hb_ref[...] = jnp.concatenate([_pack_words(h_ref[...]), zeros], axis=1)


def _prescale(h, W):
    g3, hb = pl.pallas_call(
        _scale_body,
        grid=(_R,),
        in_specs=[
            pl.BlockSpec((_N, _D), lambda r: (0, 0)),
            pl.BlockSpec((_R, _D), lambda r: (0, 0)),
        ],
        out_specs=[
            pl.BlockSpec((1, _N, _D), lambda r: (r, 0, 0)),
            pl.BlockSpec((_N, _D), lambda r: (0, 0)),
        ],
        out_shape=[
            jax.ShapeDtypeStruct((_R, _N, _D), jnp.int32),
            jax.ShapeDtypeStruct((_N, _D), jnp.int32),
        ],
    )(h, W)
    return g3.reshape(_R * _N, _D), hb


def _sc_body(g_hbm, h_hbm, idxu_hbm, idxv_hbm, out_hbm,
             idxu_v, idxv_v, outv,
             ru0, ru1, ru2, ru3, rv0, rv1, rv2, rv3,
             sem0, sem1, sem2, sem3):
    wid = lax.axis_index("s") * _NC + lax.axis_index("c")
    lane = lax.iota(jnp.int32, 16)
    rot8 = (lane + 8) % 16
    idx_c = lax.bitwise_or(lane & 8, (lane + 4) & 7)
    idx_e = lax.bitwise_or(lane & 12, (lane + 2) & 3)
    idx_g = lax.bitwise_or(lane & 14, (lane + 1) & 1)
    bitrev = ((lane & 1) * 8) + (((lane >> 1) & 1) * 4) + \
        (((lane >> 2) & 1) * 2) + ((lane >> 3) & 1)
    m8 = lane < 8
    m4 = (lane & 4) == 0
    m2 = (lane & 2) == 0
    m1 = (lane & 1) == 0
    rubufs = (ru0, ru1, ru2, ru3)
    rvbufs = (rv0, rv1, rv2, rv3)
    sems = (sem0, sem1, sem2, sem3)

    base = wid * _EPW
    pltpu.sync_copy(idxu_hbm.at[pl.ds(base, _EPW)], idxu_v)
    pltpu.sync_copy(idxv_hbm.at[pl.ds(base, _EPW)], idxv_v)

    def fire(g, b):
        off = g * _C
        pltpu.async_copy(g_hbm.at[idxu_v.at[pl.ds(off, _C)]], rubufs[b],
                         sems[b])
        pltpu.async_copy(h_hbm.at[idxv_v.at[pl.ds(off, _C)]], rvbufs[b],
                         sems[b])

    def drain(b):
        pltpu.make_async_copy(g_hbm.at[idxu_v.at[pl.ds(0, _C)]], rubufs[b],
                              sems[b]).wait()
        pltpu.make_async_copy(h_hbm.at[idxv_v.at[pl.ds(0, _C)]], rvbufs[b],
                              sems[b]).wait()

    def split(words):
        lo = lax.bitcast_convert_type(lax.shift_left(words, 16), jnp.float32)
        hi = lax.bitcast_convert_type(words, jnp.float32)
        return lo, hi

    def compute(g, b):
        ru, rv = rubufs[b], rvbufs[b]

        def half_group(e0):
            accs = []
            for i in range(8):
                e = e0 + i
                acc_lo = jnp.zeros((16,), jnp.float32)
                acc_hi = jnp.zeros((16,), jnp.float32)
                for j in range(_DW // 16):
                    ulo, uhi = split(ru[e, pl.ds(j * 16, 16)])
                    vlo, vhi = split(rv[e, pl.ds(j * 16, 16)])
                    acc_lo = acc_lo + ulo * vlo
                    acc_hi = acc_hi + uhi * vhi
                acc = acc_lo + acc_hi
                accs.append(acc + _lane_shuffle(acc, rot8))
            bb = [jnp.where(m8, accs[2 * p], accs[2 * p + 1])
                  for p in range(4)]
            cc = [x + _lane_shuffle(x, idx_c) for x in bb]
            dd = [jnp.where(m4, cc[2 * q], cc[2 * q + 1]) for q in range(2)]
            ee = [x + _lane_shuffle(x, idx_e) for x in dd]
            ff = jnp.where(m2, ee[0], ee[1])
            return ff + _lane_shuffle(ff, idx_g)

        def group_body(t, c2):
            e0 = t * 16
            z = jnp.where(m1, half_group(e0), half_group(e0 + 8))
            outv[pl.ds(g * _C + e0, 16)] = _lane_shuffle(z, bitrev)
            return c2

        lax.fori_loop(0, _C // 16, group_body, 0)

    for b in range(_NBUF - 1):
        fire(b, b)

    def ring_body(k, carry):
        for i in range(_NBUF):
            g = k * _NBUF + i
            drain(i)
            fire(g + _NBUF - 1, (i + _NBUF - 1) % _NBUF)
            compute(g, i)
        return carry

    lax.fori_loop(0, _NLOOPS, ring_body, 0)

    for g in range(_NLOOPS * _NBUF, _NCHUNK):
        b = g % _NBUF
        drain(b)
        if g + _NBUF - 1 < _NCHUNK:
            fire(g + _NBUF - 1, (g + _NBUF - 1) % _NBUF)
        compute(g, b)

    pltpu.sync_copy(outv, out_hbm.at[pl.ds(base, _EPW)])


def kernel(h, edge_index, rel_ids, W):
    src = edge_index[0].astype(jnp.int32)
    dst = edge_index[1].astype(jnp.int32)
    rel = rel_ids.astype(jnp.int32)
    idx_u = rel * _N + src
    gtab, htab = _prescale(h, W)
    mesh = plsc.VectorSubcoreMesh(core_axis_name="c", subcore_axis_name="s")
    k = pl.kernel(
        _sc_body,
        mesh=mesh,
        out_type=jax.ShapeDtypeStruct((_E,), jnp.float32),
        scratch_types=[
            pltpu.VMEM((_EPW,), jnp.int32),
            pltpu.VMEM((_EPW,), jnp.int32),
            pltpu.VMEM((_EPW,), jnp.float32),
            pltpu.VMEM((_C, _D), jnp.int32),
            pltpu.VMEM((_C, _D), jnp.int32),
            pltpu.VMEM((_C, _D), jnp.int32),
            pltpu.VMEM((_C, _D), jnp.int32),
            pltpu.VMEM((_C, _D), jnp.int32),
            pltpu.VMEM((_C, _D), jnp.int32),
            pltpu.VMEM((_C, _D), jnp.int32),
            pltpu.VMEM((_C, _D), jnp.int32),
            pltpu.SemaphoreType.DMA,
            pltpu.SemaphoreType.DMA,
            pltpu.SemaphoreType.DMA,
            pltpu.SemaphoreType.DMA,
        ],
    )
    return k(gtab, htab, idx_u, dst)


# speedup vs baseline: 1.1466x; 1.1466x over previous
import jax
import jax.numpy as jnp
from jax import lax
from jax.experimental import pallas as pl
from jax.experimental.pallas import tpu as pltpu
from jax.experimental.pallas import tpu_sc as plsc

_N = 10000
_E = 320000
_D = 128
_DW = _D // 2
_R = 8
_NC = 2
_NS = 16
_NW = _NC * _NS
_EPW = _E // _NW
_C = 80
_NCHUNK = _EPW // _C
_NBUF = 4
_NLOOPS = (_NCHUNK - (_NBUF - 1)) // _NBUF

_GATHER_DNUMS = lax.GatherDimensionNumbers(
    offset_dims=(), collapsed_slice_dims=(0,), start_index_map=(0,))


def _lane_shuffle(v, idx):
    return lax.gather(v, idx[:, None], _GATHER_DNUMS, slice_sizes=(1,),
                      mode=lax.GatherScatterMode.PROMISE_IN_BOUNDS)


def _bf16_bits(x):
    u = lax.bitcast_convert_type(x, jnp.int32)
    rounded = u + 0x7FFF + lax.bitwise_and(lax.shift_right_logical(u, 16), 1)
    return lax.shift_right_logical(rounded, 16)


def _pack_words(x):
    lo = _bf16_bits(x[:, :_DW])
    hi = _bf16_bits(x[:, _DW:])
    return lax.bitwise_or(lo, lax.shift_left(hi, 16))


def _scale_body(h_ref, w_ref, g_ref, hb_ref):
    r = pl.program_id(0)
    zeros = jnp.zeros((_N, _DW), jnp.int32)
    g = h_ref[...] * w_ref[pl.ds(r, 1), :]
    g_ref[...] = jnp.concatenate([_pack_words(g), zeros], axis=1)[None]
    hb_ref[...] = jnp.concatenate([_pack_words(h_ref[...]), zeros], axis=1)


def _prescale(h, W):
    g3, hb = pl.pallas_call(
        _scale_body,
        grid=(_R,),
        in_specs=[
            pl.BlockSpec((_N, _D), lambda r: (0, 0)),
            pl.BlockSpec((_R, _D), lambda r: (0, 0)),
        ],
        out_specs=[
            pl.BlockSpec((1, _N, _D), lambda r: (r, 0, 0)),
            pl.BlockSpec((_N, _D), lambda r: (0, 0)),
        ],
        out_shape=[
            jax.ShapeDtypeStruct((_R, _N, _D), jnp.int32),
            jax.ShapeDtypeStruct((_N, _D), jnp.int32),
        ],
    )(h, W)
    return g3.reshape(_R * _N, _D), hb


def _sc_body(g_hbm, h_hbm, idxu_hbm, idxv_hbm, out_hbm,
             idxu_v, idxv_v, outv,
             ru0, ru1, ru2, ru3, rv0, rv1, rv2, rv3,
             sem0, sem1, sem2, sem3):
    wid = lax.axis_index("s") * _NC + lax.axis_index("c")
    lane = lax.iota(jnp.int32, 16)
    shuf = [(lane + sh) % 16 for sh in (8, 4, 2, 1)]
    rubufs = (ru0, ru1, ru2, ru3)
    rvbufs = (rv0, rv1, rv2, rv3)
    sems = (sem0, sem1, sem2, sem3)

    base = wid * _EPW
    pltpu.sync_copy(idxu_hbm.at[pl.ds(base, _EPW)], idxu_v)
    pltpu.sync_copy(idxv_hbm.at[pl.ds(base, _EPW)], idxv_v)

    def fire(g, b):
        off = g * _C
        pltpu.async_copy(g_hbm.at[idxu_v.at[pl.ds(off, _C)]], rubufs[b],
                         sems[b])
        pltpu.async_copy(h_hbm.at[idxv_v.at[pl.ds(off, _C)]], rvbufs[b],
                         sems[b])

    def drain(b):
        pltpu.make_async_copy(g_hbm.at[idxu_v.at[pl.ds(0, _C)]], rubufs[b],
                              sems[b]).wait()
        pltpu.make_async_copy(h_hbm.at[idxv_v.at[pl.ds(0, _C)]], rvbufs[b],
                              sems[b]).wait()

    def split(words):
        lo = lax.bitcast_convert_type(lax.shift_left(words, 16), jnp.float32)
        hi = lax.bitcast_convert_type(words, jnp.float32)
        return lo, hi

    def compute(g, b):
        ru, rv = rubufs[b], rvbufs[b]

        def group_body(t, c2):
            e0 = t * 16
            scores = jnp.zeros((16,), jnp.float32)
            for i in range(16):
                e = e0 + i
                acc_lo = jnp.zeros((16,), jnp.float32)
                acc_hi = jnp.zeros((16,), jnp.float32)
                for j in range(_DW // 16):
                    ulo, uhi = split(ru[e, pl.ds(j * 16, 16)])
                    vlo, vhi = split(rv[e, pl.ds(j * 16, 16)])
                    acc_lo = acc_lo + ulo * vlo
                    acc_hi = acc_hi + uhi * vhi
                acc = acc_lo + acc_hi
                for s in shuf:
                    acc = acc + _lane_shuffle(acc, s)
                scores = jnp.where(lane == i, acc, scores)
            outv[pl.ds(g * _C + e0, 16)] = scores
            return c2

        lax.fori_loop(0, _C // 16, group_body, 0)

    for b in range(_NBUF - 1):
        fire(b, b)

    def ring_body(k, carry):
        for i in range(_NBUF):
            g = k * _NBUF + i
            drain(i)
            fire(g + _NBUF - 1, (i + _NBUF - 1) % _NBUF)
            compute(g, i)
        return carry

    lax.fori_loop(0, _NLOOPS, ring_body, 0)

    for g in range(_NLOOPS * _NBUF, _NCHUNK):
        b = g % _NBUF
        drain(b)
        if g + _NBUF - 1 < _NCHUNK:
            fire(g + _NBUF - 1, (g + _NBUF - 1) % _NBUF)
        compute(g, b)

    pltpu.sync_copy(outv, out_hbm.at[pl.ds(base, _EPW)])


def kernel(h, edge_index, rel_ids, W):
    src = edge_index[0].astype(jnp.int32)
    dst = edge_index[1].astype(jnp.int32)
    rel = rel_ids.astype(jnp.int32)
    idx_u = rel * _N + src
    gtab, htab = _prescale(h, W)
    mesh = plsc.VectorSubcoreMesh(core_axis_name="c", subcore_axis_name="s")
    k = pl.kernel(
        _sc_body,
        mesh=mesh,
        out_type=jax.ShapeDtypeStruct((_E,), jnp.float32),
        scratch_types=[
            pltpu.VMEM((_EPW,), jnp.int32),
            pltpu.VMEM((_EPW,), jnp.int32),
            pltpu.VMEM((_EPW,), jnp.float32),
            pltpu.VMEM((_C, _D), jnp.int32),
            pltpu.VMEM((_C, _D), jnp.int32),
            pltpu.VMEM((_C, _D), jnp.int32),
            pltpu.VMEM((_C, _D), jnp.int32),
            pltpu.VMEM((_C, _D), jnp.int32),
            pltpu.VMEM((_C, _D), jnp.int32),
            pltpu.VMEM((_C, _D), jnp.int32),
            pltpu.VMEM((_C, _D), jnp.int32),
            pltpu.SemaphoreType.DMA,
            pltpu.SemaphoreType.DMA,
            pltpu.SemaphoreType.DMA,
            pltpu.SemaphoreType.DMA,
        ],
    )
    return k(gtab, htab, idx_u, dst)
